# Initial kernel scaffold; baseline (speedup 1.0000x reference)
import jax
import jax.numpy as jnp
from jax.experimental import pallas as pl


def kernel(features, cluster_centers, pseudo_assignment):
    raise NotImplementedError("write your pallas kernel here")


# speedup vs baseline: 6.3269x; 6.3269x over previous
import functools

import jax
import jax.numpy as jnp
from jax import lax
from jax.experimental import pallas as pl
from jax.experimental.pallas import tpu as pltpu
from jax.experimental.pallas import tpu_sc as plsc

_B = 128
_HW = 1024
_DIM = 32
_K = 512
_N = _B * _HW

_TILES = 32
_R = _N // _TILES


def _argmax_body(x_ref, c_ref, lab_ref):
    x = x_ref[0]
    nrm = jnp.maximum(jnp.sqrt(jnp.sum(x * x, axis=1, keepdims=True)), 1e-12)
    xn = x / nrm
    sim = lax.dot_general(xn.astype(jnp.bfloat16),
                          c_ref[...].astype(jnp.bfloat16),
                          (((1,), (1,)), ((), ())),
                          preferred_element_type=jnp.float32)
    col = lax.broadcasted_iota(jnp.int32, sim.shape, 1)
    m = jnp.max(sim, axis=1, keepdims=True)
    lab = jnp.min(jnp.where(sim == m, col, _K), axis=1)
    lab_ref[0, 0] = lab.astype(jnp.int32)


_argmax_call = pl.pallas_call(
    _argmax_body,
    grid=(_TILES,),
    in_specs=[
        pl.BlockSpec((1, _R, _DIM), lambda i: (i, 0, 0)),
        pl.BlockSpec((_K, _DIM), lambda i: (0, 0)),
    ],
    out_specs=pl.BlockSpec((1, 1, _R), lambda i: (i, 0, 0)),
    out_shape=jax.ShapeDtypeStruct((_TILES, 1, _R), jnp.int32),
)

_NC = 2
_NS = 16
_NW = _NC * _NS
_CHUNK = _N // _NW
_L = 16


def _sc_gather_body(labels_hbm, table_hbm, out_hbm, table_v, idx_v, out_v):
    wid = lax.axis_index("s") * _NC + lax.axis_index("c")
    base = wid * _CHUNK
    pltpu.sync_copy(table_hbm, table_v)
    pltpu.sync_copy(labels_hbm.at[pl.ds(base, _CHUNK)], idx_v)

    def body(i, carry):
        sl = pl.ds(i * _L, _L)
        out_v[sl] = plsc.load_gather(table_v, [idx_v[sl]])
        return carry

    lax.fori_loop(0, _CHUNK // _L, body, 0)
    pltpu.sync_copy(out_v, out_hbm.at[pl.ds(base, _CHUNK)])


@functools.cache
def _sc_gather():
    return pl.kernel(
        _sc_gather_body,
        out_type=jax.ShapeDtypeStruct((_N,), jnp.int32),
        mesh=plsc.VectorSubcoreMesh(core_axis_name="c", subcore_axis_name="s",
                                    num_cores=_NC, num_subcores=_NS),
        compiler_params=pltpu.CompilerParams(needs_layout_passes=False),
        scratch_types=[
            pltpu.VMEM((_K,), jnp.int32),
            pltpu.VMEM((_CHUNK,), jnp.int32),
            pltpu.VMEM((_CHUNK,), jnp.int32),
        ],
    )


def kernel(features, cluster_centers, pseudo_assignment):
    feats3 = features.reshape(_TILES, _R, _DIM)
    labels = _argmax_call(feats3, cluster_centers)
    pseudo_segs_pred = labels.reshape(_B, _HW)
    segs_pred = _sc_gather()(labels.reshape(_N),
                             pseudo_assignment.astype(jnp.int32))
    return pseudo_segs_pred, segs_pred.reshape(_B, _HW)


# speedup vs baseline: 11.1727x; 1.7659x over previous
import functools

import jax
import jax.numpy as jnp
from jax import lax
from jax.experimental import pallas as pl
from jax.experimental.pallas import tpu as pltpu
from jax.experimental.pallas import tpu_sc as plsc

_B = 128
_HW = 1024
_DIM = 32
_K = 512
_N = _B * _HW

_TILES = 32
_R = _N // _TILES


def _argmax_body(x_ref, c_ref, lab_ref):
    x = x_ref[0]
    nrm = jnp.maximum(jnp.sqrt(jnp.sum(x * x, axis=1, keepdims=True)), 1e-12)
    xn = x / nrm
    sim = lax.dot_general(c_ref[...].astype(jnp.bfloat16),
                          xn.astype(jnp.bfloat16),
                          (((1,), (1,)), ((), ())),
                          preferred_element_type=jnp.float32)
    lab = jnp.argmax(sim, axis=0)
    lab_ref[0, 0] = lab.astype(jnp.int32)


_argmax_call = pl.pallas_call(
    _argmax_body,
    grid=(_TILES,),
    in_specs=[
        pl.BlockSpec((1, _R, _DIM), lambda i: (i, 0, 0)),
        pl.BlockSpec((_K, _DIM), lambda i: (0, 0)),
    ],
    out_specs=pl.BlockSpec((1, 1, _R), lambda i: (i, 0, 0)),
    out_shape=jax.ShapeDtypeStruct((_TILES, 1, _R), jnp.int32),
)

_NC = 2
_NS = 16
_NW = _NC * _NS
_CHUNK = _N // _NW
_L = 16


def _sc_gather_body(labels_hbm, table_hbm, out_hbm, table_v, idx_v, out_v):
    wid = lax.axis_index("s") * _NC + lax.axis_index("c")
    base = wid * _CHUNK
    pltpu.sync_copy(table_hbm, table_v)
    pltpu.sync_copy(labels_hbm.at[pl.ds(base, _CHUNK)], idx_v)

    def body(i, carry):
        sl = pl.ds(i * _L, _L)
        out_v[sl] = plsc.load_gather(table_v, [idx_v[sl]])
        return carry

    lax.fori_loop(0, _CHUNK // _L, body, 0)
    pltpu.sync_copy(out_v, out_hbm.at[pl.ds(base, _CHUNK)])


@functools.cache
def _sc_gather():
    return pl.kernel(
        _sc_gather_body,
        out_type=jax.ShapeDtypeStruct((_N,), jnp.int32),
        mesh=plsc.VectorSubcoreMesh(core_axis_name="c", subcore_axis_name="s",
                                    num_cores=_NC, num_subcores=_NS),
        compiler_params=pltpu.CompilerParams(needs_layout_passes=False),
        scratch_types=[
            pltpu.VMEM((_K,), jnp.int32),
            pltpu.VMEM((_CHUNK,), jnp.int32),
            pltpu.VMEM((_CHUNK,), jnp.int32),
        ],
    )


def kernel(features, cluster_centers, pseudo_assignment):
    feats3 = features.reshape(_TILES, _R, _DIM)
    labels = _argmax_call(feats3, cluster_centers)
    pseudo_segs_pred = labels.reshape(_B, _HW)
    segs_pred = _sc_gather()(labels.reshape(_N),
                             pseudo_assignment.astype(jnp.int32))
    return pseudo_segs_pred, segs_pred.reshape(_B, _HW)


# speedup vs baseline: 11.3113x; 1.0124x over previous
import functools

import jax
import jax.numpy as jnp
from jax import lax
from jax.experimental import pallas as pl
from jax.experimental.pallas import tpu as pltpu
from jax.experimental.pallas import tpu_sc as plsc

_B = 128
_HW = 1024
_DIM = 32
_K = 512
_N = _B * _HW

_TILES = 16
_R = _N // _TILES
_RB = _R // _HW


def _argmax_body(x_ref, c_ref, lab_ref):
    x = x_ref[0]
    nrm = jnp.maximum(jnp.sqrt(jnp.sum(x * x, axis=1, keepdims=True)), 1e-12)
    xn = x / nrm
    sim = lax.dot_general(c_ref[...].astype(jnp.bfloat16),
                          xn.astype(jnp.bfloat16),
                          (((1,), (1,)), ((), ())),
                          preferred_element_type=jnp.float32)
    lab = jnp.argmax(sim, axis=0)
    lab_ref[0] = lab.astype(jnp.int32).reshape(_RB, _HW)


_argmax_call = pl.pallas_call(
    _argmax_body,
    grid=(_TILES,),
    in_specs=[
        pl.BlockSpec((1, _R, _DIM), lambda i: (i, 0, 0)),
        pl.BlockSpec((_K, _DIM), lambda i: (0, 0)),
    ],
    out_specs=pl.BlockSpec((1, _RB, _HW), lambda i: (i, 0, 0)),
    out_shape=jax.ShapeDtypeStruct((_TILES, _RB, _HW), jnp.int32),
)

_NC = 2
_NS = 16
_NW = _NC * _NS
_ROWS_W = _B // _NW
_CHUNK = _ROWS_W * _HW
_L = 16


def _sc_gather_body(labels_hbm, table_hbm, out_hbm, table_v, idx_v, out_v):
    wid = lax.axis_index("s") * _NC + lax.axis_index("c")
    row0 = wid * _ROWS_W
    pltpu.sync_copy(table_hbm, table_v)
    for j in range(_ROWS_W):
        pltpu.sync_copy(labels_hbm.at[row0 + j],
                        idx_v.at[pl.ds(j * _HW, _HW)])

    def body(i, carry):
        sl = pl.ds(i * _L, _L)
        out_v[sl] = plsc.load_gather(table_v, [idx_v[sl]])
        return carry

    lax.fori_loop(0, _CHUNK // _L, body, 0)
    for j in range(_ROWS_W):
        pltpu.sync_copy(out_v.at[pl.ds(j * _HW, _HW)],
                        out_hbm.at[row0 + j])


@functools.cache
def _sc_gather():
    return pl.kernel(
        _sc_gather_body,
        out_type=jax.ShapeDtypeStruct((_B, _HW), jnp.int32),
        mesh=plsc.VectorSubcoreMesh(core_axis_name="c", subcore_axis_name="s",
                                    num_cores=_NC, num_subcores=_NS),
        compiler_params=pltpu.CompilerParams(needs_layout_passes=False),
        scratch_types=[
            pltpu.VMEM((_K,), jnp.int32),
            pltpu.VMEM((_CHUNK,), jnp.int32),
            pltpu.VMEM((_CHUNK,), jnp.int32),
        ],
    )


def kernel(features, cluster_centers, pseudo_assignment):
    feats3 = features.reshape(_TILES, _R, _DIM)
    labels = _argmax_call(feats3, cluster_centers)
    pseudo_segs_pred = labels.reshape(_B, _HW)
    segs_pred = _sc_gather()(pseudo_segs_pred,
                             pseudo_assignment.astype(jnp.int32))
    return pseudo_segs_pred, segs_pred


# speedup vs baseline: 19.5072x; 1.7246x over previous
import functools

import jax
import jax.numpy as jnp
from jax import lax
from jax.experimental import pallas as pl
from jax.experimental.pallas import tpu as pltpu
from jax.experimental.pallas import tpu_sc as plsc

_B = 128
_HW = 1024
_DIM = 32
_K = 512
_N = _B * _HW

_TILES = 16
_R = _N // _TILES
_RB = _R // _HW


def _argmax_body(xt_ref, ct_ref, lab_ref):
    ct = ct_ref[...].astype(jnp.bfloat16)
    for j in range(_RB):
        x = xt_ref[j]
        nrm = jnp.maximum(
            jnp.sqrt(jnp.sum(x * x, axis=0, keepdims=True)), 1e-12)
        xn = x / nrm
        sim = lax.dot_general(ct, xn.astype(jnp.bfloat16),
                              (((0,), (0,)), ((), ())),
                              preferred_element_type=jnp.float32)
        lab = jnp.argmax(sim, axis=0)
        lab_ref[j] = lab.astype(jnp.int32)


_argmax_call = pl.pallas_call(
    _argmax_body,
    grid=(_TILES,),
    in_specs=[
        pl.BlockSpec((_RB, _DIM, _HW), lambda i: (i, 0, 0)),
        pl.BlockSpec((_DIM, _K), lambda i: (0, 0)),
    ],
    out_specs=pl.BlockSpec((_RB, _HW), lambda i: (i, 0)),
    out_shape=jax.ShapeDtypeStruct((_B, _HW), jnp.int32),
)

_NC = 2
_NS = 16
_NW = _NC * _NS
_ROWS_W = _B // _NW
_CHUNK = _ROWS_W * _HW
_L = 16


def _sc_gather_body(labels_hbm, table_hbm, out_hbm, table_v, idx_v, out_v):
    wid = lax.axis_index("s") * _NC + lax.axis_index("c")
    row0 = wid * _ROWS_W
    pltpu.sync_copy(table_hbm, table_v)
    for j in range(_ROWS_W):
        pltpu.sync_copy(labels_hbm.at[row0 + j],
                        idx_v.at[pl.ds(j * _HW, _HW)])

    def body(i, carry):
        sl = pl.ds(i * _L, _L)
        out_v[sl] = plsc.load_gather(table_v, [idx_v[sl]])
        return carry

    lax.fori_loop(0, _CHUNK // _L, body, 0)
    for j in range(_ROWS_W):
        pltpu.sync_copy(out_v.at[pl.ds(j * _HW, _HW)],
                        out_hbm.at[row0 + j])


@functools.cache
def _sc_gather():
    return pl.kernel(
        _sc_gather_body,
        out_type=jax.ShapeDtypeStruct((_B, _HW), jnp.int32),
        mesh=plsc.VectorSubcoreMesh(core_axis_name="c", subcore_axis_name="s",
                                    num_cores=_NC, num_subcores=_NS),
        compiler_params=pltpu.CompilerParams(needs_layout_passes=False),
        scratch_types=[
            pltpu.VMEM((_K,), jnp.int32),
            pltpu.VMEM((_CHUNK,), jnp.int32),
            pltpu.VMEM((_CHUNK,), jnp.int32),
        ],
    )


def kernel(features, cluster_centers, pseudo_assignment):
    feats_t = jnp.swapaxes(features, 1, 2)
    centers_t = cluster_centers.T
    pseudo_segs_pred = _argmax_call(feats_t, centers_t)
    segs_pred = _sc_gather()(pseudo_segs_pred,
                             pseudo_assignment.astype(jnp.int32))
    return pseudo_segs_pred, segs_pred


# speedup vs baseline: 19.8100x; 1.0155x over previous
import functools

import jax
import jax.numpy as jnp
from jax import lax
from jax.experimental import pallas as pl
from jax.experimental.pallas import tpu as pltpu
from jax.experimental.pallas import tpu_sc as plsc

_B = 128
_HW = 1024
_DIM = 32
_K = 512
_N = _B * _HW

_TILES = 16
_R = _N // _TILES
_RB = _R // _HW


def _argmax_body(xt_ref, ct_ref, lab_ref):
    ct = ct_ref[...].astype(jnp.bfloat16)
    for j in range(_RB):
        x = xt_ref[j]
        nrm = jnp.maximum(
            jnp.sqrt(jnp.sum(x * x, axis=0, keepdims=True)), 1e-12)
        xn = x / nrm
        sim = lax.dot_general(ct, xn.astype(jnp.bfloat16),
                              (((0,), (0,)), ((), ())),
                              preferred_element_type=jnp.float32)
        lab = jnp.argmax(sim, axis=0)
        lab_ref[j] = lab.astype(jnp.int32)


_argmax_call = pl.pallas_call(
    _argmax_body,
    grid=(_TILES,),
    in_specs=[
        pl.BlockSpec((_RB, _DIM, _HW), lambda i: (i, 0, 0)),
        pl.BlockSpec((_DIM, _K), lambda i: (0, 0)),
    ],
    out_specs=pl.BlockSpec((_RB, _HW), lambda i: (i, 0)),
    out_shape=jax.ShapeDtypeStruct((_B, _HW), jnp.int32),
)

_NC = 2
_NS = 16
_NW = _NC * _NS
_ROWS_W = _B // _NW
_CHUNK = _ROWS_W * _HW
_L = 16


def _sc_gather_body(labels_hbm, table_hbm, out_hbm, table_v, idx_v, out_v):
    wid = lax.axis_index("s") * _NC + lax.axis_index("c")
    row0 = wid * _ROWS_W
    pltpu.sync_copy(table_hbm, table_v)
    for j in range(_ROWS_W):
        pltpu.sync_copy(labels_hbm.at[row0 + j],
                        idx_v.at[pl.ds(j * _HW, _HW)])

    @plsc.parallel_loop(0, _CHUNK, step=_L, unroll=8)
    def body(i):
        sl = pl.ds(i, _L)
        out_v[sl] = plsc.load_gather(table_v, [idx_v[sl]])
    for j in range(_ROWS_W):
        pltpu.sync_copy(out_v.at[pl.ds(j * _HW, _HW)],
                        out_hbm.at[row0 + j])


@functools.cache
def _sc_gather():
    return pl.kernel(
        _sc_gather_body,
        out_type=jax.ShapeDtypeStruct((_B, _HW), jnp.int32),
        mesh=plsc.VectorSubcoreMesh(core_axis_name="c", subcore_axis_name="s",
                                    num_cores=_NC, num_subcores=_NS),
        compiler_params=pltpu.CompilerParams(needs_layout_passes=False),
        scratch_types=[
            pltpu.VMEM((_K,), jnp.int32),
            pltpu.VMEM((_CHUNK,), jnp.int32),
            pltpu.VMEM((_CHUNK,), jnp.int32),
        ],
    )


def kernel(features, cluster_centers, pseudo_assignment):
    feats_t = jnp.swapaxes(features, 1, 2)
    centers_t = cluster_centers.T
    pseudo_segs_pred = _argmax_call(feats_t, centers_t)
    segs_pred = _sc_gather()(pseudo_segs_pred,
                             pseudo_assignment.astype(jnp.int32))
    return pseudo_segs_pred, segs_pred


# speedup vs baseline: 20.3929x; 1.0294x over previous
import functools

import jax
import jax.numpy as jnp
from jax import lax
from jax.experimental import pallas as pl
from jax.experimental.pallas import tpu as pltpu
from jax.experimental.pallas import tpu_sc as plsc

_B = 128
_HW = 1024
_DIM = 32
_K = 512
_N = _B * _HW

_TILES = 8
_R = _N // _TILES
_RB = _R // _HW


def _argmax_body(xt_ref, ct_ref, lab_ref):
    ct = ct_ref[...].astype(jnp.bfloat16)
    for j in range(_RB):
        x = xt_ref[j]
        nrm = jnp.maximum(
            jnp.sqrt(jnp.sum(x * x, axis=0, keepdims=True)), 1e-12)
        xn = x / nrm
        sim = lax.dot_general(ct, xn.astype(jnp.bfloat16),
                              (((0,), (0,)), ((), ())),
                              preferred_element_type=jnp.float32)
        lab = jnp.argmax(sim, axis=0)
        lab_ref[j] = lab.astype(jnp.int32)


_argmax_call = pl.pallas_call(
    _argmax_body,
    grid=(_TILES,),
    in_specs=[
        pl.BlockSpec((_RB, _DIM, _HW), lambda i: (i, 0, 0)),
        pl.BlockSpec((_DIM, _K), lambda i: (0, 0)),
    ],
    out_specs=pl.BlockSpec((_RB, _HW), lambda i: (i, 0)),
    out_shape=jax.ShapeDtypeStruct((_B, _HW), jnp.int32),
)

_NC = 2
_NS = 16
_NW = _NC * _NS
_ROWS_W = _B // _NW
_CHUNK = _ROWS_W * _HW
_L = 16


def _sc_gather_body(labels_hbm, table_hbm, out_hbm, table_v, idx_v, out_v):
    wid = lax.axis_index("s") * _NC + lax.axis_index("c")
    row0 = wid * _ROWS_W
    pltpu.sync_copy(table_hbm, table_v)
    for j in range(_ROWS_W):
        pltpu.sync_copy(labels_hbm.at[row0 + j],
                        idx_v.at[pl.ds(j * _HW, _HW)])

    @plsc.parallel_loop(0, _CHUNK, step=_L, unroll=8)
    def body(i):
        sl = pl.ds(i, _L)
        out_v[sl] = plsc.load_gather(table_v, [idx_v[sl]])
    for j in range(_ROWS_W):
        pltpu.sync_copy(out_v.at[pl.ds(j * _HW, _HW)],
                        out_hbm.at[row0 + j])


@functools.cache
def _sc_gather():
    return pl.kernel(
        _sc_gather_body,
        out_type=jax.ShapeDtypeStruct((_B, _HW), jnp.int32),
        mesh=plsc.VectorSubcoreMesh(core_axis_name="c", subcore_axis_name="s",
                                    num_cores=_NC, num_subcores=_NS),
        compiler_params=pltpu.CompilerParams(needs_layout_passes=False),
        scratch_types=[
            pltpu.VMEM((_K,), jnp.int32),
            pltpu.VMEM((_CHUNK,), jnp.int32),
            pltpu.VMEM((_CHUNK,), jnp.int32),
        ],
    )


def kernel(features, cluster_centers, pseudo_assignment):
    feats_t = jnp.swapaxes(features, 1, 2)
    centers_t = cluster_centers.T
    pseudo_segs_pred = _argmax_call(feats_t, centers_t)
    segs_pred = _sc_gather()(pseudo_segs_pred,
                             pseudo_assignment.astype(jnp.int32))
    return pseudo_segs_pred, segs_pred


# speedup vs baseline: 20.4776x; 1.0042x over previous
import functools

import jax
import jax.numpy as jnp
from jax import lax
from jax.experimental import pallas as pl
from jax.experimental.pallas import tpu as pltpu
from jax.experimental.pallas import tpu_sc as plsc

_B = 128
_HW = 1024
_DIM = 32
_K = 512
_N = _B * _HW

_TILES = 4
_R = _N // _TILES
_RB = _R // _HW


def _argmax_body(xt_ref, ct_ref, lab_ref):
    ct = ct_ref[...].astype(jnp.bfloat16)
    for j in range(_RB):
        x = xt_ref[j]
        nrm = jnp.maximum(
            jnp.sqrt(jnp.sum(x * x, axis=0, keepdims=True)), 1e-12)
        xn = x / nrm
        sim = lax.dot_general(ct, xn.astype(jnp.bfloat16),
                              (((0,), (0,)), ((), ())),
                              preferred_element_type=jnp.float32)
        lab = jnp.argmax(sim, axis=0)
        lab_ref[j] = lab.astype(jnp.int32)


_argmax_call = pl.pallas_call(
    _argmax_body,
    grid=(_TILES,),
    in_specs=[
        pl.BlockSpec((_RB, _DIM, _HW), lambda i: (i, 0, 0)),
        pl.BlockSpec((_DIM, _K), lambda i: (0, 0)),
    ],
    out_specs=pl.BlockSpec((_RB, _HW), lambda i: (i, 0)),
    out_shape=jax.ShapeDtypeStruct((_B, _HW), jnp.int32),
)

_NC = 2
_NS = 16
_NW = _NC * _NS
_ROWS_W = _B // _NW
_CHUNK = _ROWS_W * _HW
_L = 16


def _sc_gather_body(labels_hbm, table_hbm, out_hbm, table_v, idx_v, out_v):
    wid = lax.axis_index("s") * _NC + lax.axis_index("c")
    row0 = wid * _ROWS_W
    pltpu.sync_copy(table_hbm, table_v)
    for j in range(_ROWS_W):
        pltpu.sync_copy(labels_hbm.at[row0 + j],
                        idx_v.at[pl.ds(j * _HW, _HW)])

    @plsc.parallel_loop(0, _CHUNK, step=_L, unroll=8)
    def body(i):
        sl = pl.ds(i, _L)
        out_v[sl] = plsc.load_gather(table_v, [idx_v[sl]])
    for j in range(_ROWS_W):
        pltpu.sync_copy(out_v.at[pl.ds(j * _HW, _HW)],
                        out_hbm.at[row0 + j])


@functools.cache
def _sc_gather():
    return pl.kernel(
        _sc_gather_body,
        out_type=jax.ShapeDtypeStruct((_B, _HW), jnp.int32),
        mesh=plsc.VectorSubcoreMesh(core_axis_name="c", subcore_axis_name="s",
                                    num_cores=_NC, num_subcores=_NS),
        compiler_params=pltpu.CompilerParams(needs_layout_passes=False),
        scratch_types=[
            pltpu.VMEM((_K,), jnp.int32),
            pltpu.VMEM((_CHUNK,), jnp.int32),
            pltpu.VMEM((_CHUNK,), jnp.int32),
        ],
    )


def kernel(features, cluster_centers, pseudo_assignment):
    feats_t = jnp.swapaxes(features, 1, 2)
    centers_t = cluster_centers.T
    pseudo_segs_pred = _argmax_call(feats_t, centers_t)
    segs_pred = _sc_gather()(pseudo_segs_pred,
                             pseudo_assignment.astype(jnp.int32))
    return pseudo_segs_pred, segs_pred
